# baseline (speedup 1.0000x reference)
import dataclasses
import functools

import jax
import jax.numpy as jnp
from jax.experimental import pallas as pl
from jax.experimental.pallas import tpu as pltpu
from jax.experimental.pallas import tpu_sc as plsc

TEMP = 0.1
MARGIN = 1.0
EPS = 1e-8

N_NODES = 10000
D = 128
N_POS = 320000
N_NEG = 320000
N_ALL = N_POS + N_NEG
NPAD = 10240
G_ROWS = NPAD // 2 * NPAD // 128

GW = 640
SUB = 128
GT = 1024


def _prep_body(emb_ref, flow_ref, nemb_ref, c_ref):
    x = emb_ref[...]
    ss = jnp.sum(x * x, axis=1, keepdims=True)
    inv = 1.0 / jnp.maximum(jnp.sqrt(ss), 1e-12)
    nemb_ref[...] = (x * inv).astype(jnp.bfloat16)
    c_ref[...] = -jnp.log(flow_ref[...] + EPS) * (1.0 / TEMP)


def _prep(emb_pad, flow2d):
    return pl.pallas_call(
        _prep_body,
        out_shape=(
            jax.ShapeDtypeStruct((NPAD, D), jnp.bfloat16),
            jax.ShapeDtypeStruct((1, N_POS), jnp.float32),
        ),
    )(emb_pad, flow2d)


def _gram_body(a_ref, b_ref, o_ref):
    g = jax.lax.dot_general(
        a_ref[...], b_ref[...], (((1,), (1,)), ((), ())),
        preferred_element_type=jnp.float32,
    )
    gp = pltpu.bitcast(g.astype(jnp.bfloat16), jnp.int32)
    o_ref[...] = gp.reshape(GT // 2, GT // 128, 128)


def _gram(nemb):
    n_t = NPAD // GT
    return pl.pallas_call(
        _gram_body,
        grid=(n_t, n_t),
        in_specs=[
            pl.BlockSpec((GT, D), lambda m, n: (m, 0)),
            pl.BlockSpec((GT, D), lambda m, n: (n, 0)),
        ],
        out_specs=pl.BlockSpec((GT // 2, GT // 128, 128), lambda m, n: (m, n, 0)),
        out_shape=jax.ShapeDtypeStruct((NPAD // 2, NPAD // 128, 128), jnp.int32),
        compiler_params=pltpu.CompilerParams(
            dimension_semantics=("parallel", "parallel")
        ),
    )(nemb, nemb)


def _sc_loss(g128, i_pos, j_pos, i_neg, j_neg, c):
    mesh = plsc.VectorSubcoreMesh(core_axis_name="core", subcore_axis_name="subcore")
    cp = pltpu.CompilerParams()
    if "needs_layout_passes" in pltpu.CompilerParams.__dataclass_fields__:
        cp = dataclasses.replace(cp, needs_layout_passes=False)

    @functools.partial(
        pl.kernel,
        out_type=jax.ShapeDtypeStruct((32, 16), jnp.float32),
        mesh=mesh,
        compiler_params=cp,
        scratch_types=[
            pltpu.VMEM((GW // SUB, SUB), jnp.int32),
            pltpu.VMEM((GW,), jnp.int32),
            pltpu.VMEM((GW,), jnp.int32),
            pltpu.VMEM((GW, 128), jnp.int32),
            pltpu.VMEM((16,), jnp.float32),
            pltpu.SemaphoreType.DMA,
        ],
    )
    def k(g_hbm, ip_hbm, jp_hbm, in_hbm, jn_hbm, c_hbm, o_hbm,
          idx_v, lane_v, half_v, gwin, acc, sem):
        acc[...] = jnp.zeros((16,), jnp.float32)

        def gather_window(i_vmem, j_vmem):
            for s in range(GW // SUB):
                for cc in range(SUB // 16):
                    o = s * SUB + cc * 16
                    sl = pl.ds(o, 16)
                    iv = i_vmem[0, sl]
                    p = jax.lax.shift_right_logical(iv, 1) * NPAD + j_vmem[0, sl]
                    idx_v[s, pl.ds(cc * 16, 16)] = jax.lax.shift_right_logical(p, 7)
                    lane_v[sl] = p & 127
                    half_v[sl] = iv & 1
            cps = [
                pltpu.async_copy(
                    g_hbm.at[idx_v.at[s]], gwin.at[pl.ds(s * SUB, SUB)], sem
                )
                for s in range(GW // SUB)
            ]
            for c_ in cps:
                c_.wait()

        def select_sims(cc):
            sl = pl.ds(cc * 16, 16)
            rows = jax.lax.iota(jnp.int32, 16) + (cc * 16)
            w = plsc.load_gather(gwin, [rows, lane_v[sl]])
            bits = jnp.where(
                half_v[sl] == 0,
                jax.lax.shift_left(w, 16),
                w & jnp.int32(-65536),
            )
            return plsc.bitcast(bits, jnp.float32), sl

        def pos_body(i_vmem, j_vmem, c_vmem):
            gather_window(i_vmem, j_vmem)
            for cc in range(GW // 16):
                sims, sl = select_sims(cc)
                acc[...] = acc[...] + sims * c_vmem[0, sl]

        def neg_body(i_vmem, j_vmem):
            gather_window(i_vmem, j_vmem)
            for cc in range(GW // 16):
                sims, _ = select_sims(cc)
                acc[...] = acc[...] + jnp.maximum(
                    sims * (1.0 / TEMP) - MARGIN, 0.0
                )

        pairspec = pl.BlockSpec((1, GW), lambda w: (0, w))
        pltpu.emit_pipeline(
            pos_body,
            grid=(N_POS // GW,),
            in_specs=[pairspec, pairspec, pairspec],
            out_specs=[],
            core_axis_name=("core", "subcore"),
            dimension_semantics=(pltpu.PARALLEL,),
        )(ip_hbm, jp_hbm, c_hbm)
        pltpu.emit_pipeline(
            neg_body,
            grid=(N_NEG // GW,),
            in_specs=[pairspec, pairspec],
            out_specs=[],
            core_axis_name=("core", "subcore"),
            dimension_semantics=(pltpu.PARALLEL,),
        )(in_hbm, jn_hbm)

        wid = jax.lax.axis_index("core") * 16 + jax.lax.axis_index("subcore")
        pltpu.sync_copy(acc, o_hbm.at[wid])

    return k(g128, i_pos, j_pos, i_neg, j_neg, c)


def kernel(embeddings, positive_pairs, flow_weights, negative_pairs):
    emb_pad = jnp.pad(embeddings, ((0, NPAD - N_NODES), (0, 0)))
    nemb, c = _prep(emb_pad, flow_weights.reshape(1, N_POS))
    g = _gram(nemb)
    g128 = g.reshape(G_ROWS, 128)
    partials = _sc_loss(
        g128,
        positive_pairs[0].astype(jnp.int32).reshape(1, N_POS),
        positive_pairs[1].astype(jnp.int32).reshape(1, N_POS),
        negative_pairs[0].astype(jnp.int32).reshape(1, N_NEG),
        negative_pairs[1].astype(jnp.int32).reshape(1, N_NEG),
        c,
    )
    return jnp.sum(partials) * (1.0 / N_ALL)
